# Initial kernel scaffold; baseline (speedup 1.0000x reference)
import jax
import jax.numpy as jnp
from jax.experimental import pallas as pl


def kernel(x, edge_index, edge_attr, W1, b1, W2, b2, Wi, bi, Wo, bo):
    raise NotImplementedError("write your pallas kernel here")


# speedup vs baseline: 1.2661x; 1.2661x over previous
import jax
import jax.numpy as jnp
from jax.experimental import pallas as pl
from jax.experimental.pallas import tpu as pltpu

N = 10000
NPAD = 10240


def _head_body(h_ref, wi_ref, bi_ref, wo_ref, bo_ref, o_ref):
    h = h_ref[...]
    t = jnp.maximum(jnp.dot(h, wi_ref[...], preferred_element_type=jnp.float32)
                    + bi_ref[...], 0.0)
    o_ref[...] = jnp.dot(t, wo_ref[...], preferred_element_type=jnp.float32) + bo_ref[...]


def _head(h, Wi, bi, Wo, bo):
    npad = h.shape[0]
    blk = 2048
    grid = npad // blk
    return pl.pallas_call(
        _head_body,
        grid=(grid,),
        in_specs=[
            pl.BlockSpec((blk, h.shape[1]), lambda i: (i, 0)),
            pl.BlockSpec(Wi.shape, lambda i: (0, 0)),
            pl.BlockSpec(bi.shape, lambda i: (0,)),
            pl.BlockSpec(Wo.shape, lambda i: (0, 0)),
            pl.BlockSpec(bo.shape, lambda i: (0,)),
        ],
        out_specs=pl.BlockSpec((blk, 1), lambda i: (i, 0)),
        out_shape=jax.ShapeDtypeStruct((npad, 1), jnp.float32),
    )(h, Wi, bi, Wo, bo)


def _gcn(x, row, col, w, deg_inv_sqrt, W, b):
    xw = x @ W
    norm = deg_inv_sqrt[row] * w * deg_inv_sqrt[col]
    msg = xw[row] * norm[:, None]
    out = jnp.zeros((N, W.shape[1]), dtype=x.dtype).at[col].add(msg)
    out = out + (deg_inv_sqrt * deg_inv_sqrt)[:, None] * xw
    return out + b


def kernel(x, edge_index, edge_attr, W1, b1, W2, b2, Wi, bi, Wo, bo):
    row = edge_index[0]
    col = edge_index[1]
    deg = jnp.ones((N,), jnp.float32).at[col].add(edge_attr)
    dinv = jax.lax.rsqrt(deg)
    h = jax.nn.relu(_gcn(x, row, col, edge_attr, dinv, W1, b1))
    h = jax.nn.relu(_gcn(h, row, col, edge_attr, dinv, W2, b2))
    hp = jnp.pad(h, ((0, NPAD - N), (0, 0)))
    out = _head(hp, Wi, bi, Wo, bo)
    return out[:N]


# speedup vs baseline: 6.1551x; 4.8614x over previous
import functools

import jax
import jax.numpy as jnp
from jax import lax
from jax.experimental import pallas as pl
from jax.experimental.pallas import tpu as pltpu
from jax.experimental.pallas import tpu_sc as plsc

N = 10000
NPAD = 10240
E = 320000
NC = 2
NS = 16
CH = 80
WB = 80
EPT = E // NS
NPT = NPAD // NS

_mesh = plsc.VectorSubcoreMesh(core_axis_name="c", subcore_axis_name="s")


def _deg_body(col_hbm, w_hbm, zeros_hbm, deg_hbm, colv, wv, w16, obuf, deg_sh):
    c = lax.axis_index("c")
    s = lax.axis_index("s")
    r0 = s * NPT
    pltpu.sync_copy(zeros_hbm.at[pl.ds(r0, NPT)], deg_sh.at[pl.ds(r0, NPT)])
    plsc.subcore_barrier()

    base0 = c * (E // NC) + s * (E // NC // NS)
    nchunks = E // NC // NS // CH

    def chunk(i, _):
        base = base0 + i * CH
        pltpu.sync_copy(col_hbm.at[pl.ds(base, CH)], colv.at[0])
        pltpu.sync_copy(w_hbm.at[pl.ds(base, CH)], wv)
        for g in range(CH // 16):
            wvec = wv[pl.ds(g * 16, 16)]
            for kk in range(16):
                w16[g * 16 + kk, pl.ds(0, 16)] = jnp.broadcast_to(wvec[kk], (16,))
        pltpu.sync_copy(w16, deg_sh.at[colv.at[0]], add=True)
        return _

    lax.fori_loop(0, nchunks, chunk, None)
    plsc.subcore_barrier()

    for t in range(NPT // 128):
        pltpu.sync_copy(deg_sh.at[pl.ds(r0 + t * 128, 128)], obuf)
        pltpu.sync_copy(obuf, deg_hbm.at[c].at[pl.ds(r0 + t * 128, 128)])


def _deg(col, w, zeros):
    f = functools.partial(
        pl.kernel,
        out_type=jax.ShapeDtypeStruct((NC, NPAD, 128), jnp.float32),
        mesh=_mesh,
        scratch_types=[
            pltpu.VMEM((1, CH), jnp.int32),
            pltpu.VMEM((CH,), jnp.float32),
            pltpu.VMEM((CH, 128), jnp.float32),
            pltpu.VMEM((128, 128), jnp.float32),
            pltpu.VMEM_SHARED((NPAD, 128), jnp.float32),
        ],
    )(_deg_body)
    return f(col, w, zeros)


def _layer_body(y_hbm, row_hbm, col_hbm, w_hbm, dinv_hbm, b_hbm, zeros_hbm,
                h_hbm, rowv, colv, wv, buf, abuf, ybuf, hbuf, dvbuf, bbuf,
                sem, acc_sh):
    c = lax.axis_index("c")
    s = lax.axis_index("s")
    r0 = s * NPT
    pltpu.sync_copy(zeros_hbm.at[pl.ds(r0, NPT)], acc_sh.at[pl.ds(r0, NPT)])
    pltpu.sync_copy(b_hbm.at[c], bbuf)
    plsc.subcore_barrier()

    base0 = s * EPT
    yc = y_hbm.at[c]

    def chunk(i, _):
        base = base0 + i * CH
        pltpu.sync_copy(row_hbm.at[pl.ds(base, CH)], rowv)
        pltpu.sync_copy(col_hbm.at[pl.ds(base, CH)], colv.at[0])
        pltpu.sync_copy(w_hbm.at[pl.ds(base, CH)], wv)
        pltpu.async_copy(yc.at[rowv], buf, sem).wait()

        def edge_group(g, _):
            wvec = wv[pl.ds(g * 16, 16)]
            for kk in range(16):
                bw = jnp.broadcast_to(wvec[kk], (16,))
                for j in range(8):
                    sl = pl.ds(j * 16, 16)
                    buf[g * 16 + kk, sl] = buf[g * 16 + kk, sl] * bw
            return _

        lax.fori_loop(0, CH // 16, edge_group, None)
        pltpu.sync_copy(buf, acc_sh.at[colv.at[0]], add=True)
        return _

    lax.fori_loop(0, EPT // CH, chunk, None)
    plsc.subcore_barrier()

    for t in range(NPT // WB):
        w0 = r0 + t * WB
        pltpu.sync_copy(acc_sh.at[pl.ds(w0, WB)], abuf)
        pltpu.sync_copy(yc.at[pl.ds(w0, WB)], ybuf)
        pltpu.sync_copy(dinv_hbm.at[pl.ds(w0, WB)], dvbuf)

        def node_group(g, _):
            dvec = dvbuf[pl.ds(g * 16, 16)]
            for kk in range(16):
                dv = jnp.broadcast_to(dvec[kk], (16,))
                for j in range(8):
                    sl = pl.ds(j * 16, 16)
                    hbuf[g * 16 + kk, sl] = jnp.maximum(
                        (abuf[g * 16 + kk, sl] + ybuf[g * 16 + kk, sl]) * dv
                        + bbuf[sl], 0.0)
            return _

        lax.fori_loop(0, WB // 16, node_group, None)
        pltpu.sync_copy(hbuf, h_hbm.at[c].at[pl.ds(w0, WB)])


def _layer(y, row, col, w, dinv, bh, zeros):
    f = functools.partial(
        pl.kernel,
        out_type=jax.ShapeDtypeStruct((NC, NPAD, 128), jnp.float32),
        mesh=_mesh,
        scratch_types=[
            pltpu.VMEM((CH,), jnp.int32),
            pltpu.VMEM((1, CH), jnp.int32),
            pltpu.VMEM((CH,), jnp.float32),
            pltpu.VMEM((CH, 128), jnp.float32),
            pltpu.VMEM((WB, 128), jnp.float32),
            pltpu.VMEM((WB, 128), jnp.float32),
            pltpu.VMEM((WB, 128), jnp.float32),
            pltpu.VMEM((WB,), jnp.float32),
            pltpu.VMEM((128,), jnp.float32),
            pltpu.SemaphoreType.DMA,
            pltpu.VMEM_SHARED((NPAD, 128), jnp.float32),
        ],
    )(_layer_body)
    return f(y, row, col, w, dinv, bh, zeros)


def _mm1_body(x_ref, wh_ref, d0_ref, d1_ref, y_ref, dinv_ref):
    deg = d0_ref[0, :, :1] + d1_ref[0, :, :1] + 1.0
    dinv = lax.rsqrt(deg)
    x = x_ref[...]
    y_ref[0] = jnp.dot(x, wh_ref[0], preferred_element_type=jnp.float32) * dinv
    y_ref[1] = jnp.dot(x, wh_ref[1], preferred_element_type=jnp.float32) * dinv
    dinv_ref[...] = dinv[:, 0]


def _mm1(xp, Wh, degp):
    blk = 1024
    return pl.pallas_call(
        _mm1_body,
        grid=(NPAD // blk,),
        in_specs=[
            pl.BlockSpec((blk, 128), lambda i: (i, 0)),
            pl.BlockSpec((2, 128, 128), lambda i: (0, 0, 0)),
            pl.BlockSpec((1, blk, 128), lambda i: (0, i, 0)),
            pl.BlockSpec((1, blk, 128), lambda i: (1, i, 0)),
        ],
        out_specs=[
            pl.BlockSpec((2, blk, 128), lambda i: (0, i, 0)),
            pl.BlockSpec((blk,), lambda i: (i,)),
        ],
        out_shape=[
            jax.ShapeDtypeStruct((NC, NPAD, 128), jnp.float32),
            jax.ShapeDtypeStruct((NPAD,), jnp.float32),
        ],
    )(xp, Wh, degp, degp)


def _mm2_body(h0_ref, h1_ref, w_ref, dinv_ref, y_ref):
    dinv = dinv_ref[...][:, None]
    h0 = h0_ref[0]
    h1 = h1_ref[0]
    for c in range(2):
        xw = (jnp.dot(h0, w_ref[0, c], preferred_element_type=jnp.float32)
              + jnp.dot(h1, w_ref[1, c], preferred_element_type=jnp.float32))
        y_ref[c] = xw * dinv


def _mm2(h, W2q, dinv):
    blk = 1024
    return pl.pallas_call(
        _mm2_body,
        grid=(NPAD // blk,),
        in_specs=[
            pl.BlockSpec((1, blk, 128), lambda i: (0, i, 0)),
            pl.BlockSpec((1, blk, 128), lambda i: (1, i, 0)),
            pl.BlockSpec((2, 2, 128, 128), lambda i: (0, 0, 0, 0)),
            pl.BlockSpec((blk,), lambda i: (i,)),
        ],
        out_specs=pl.BlockSpec((2, blk, 128), lambda i: (0, i, 0)),
        out_shape=jax.ShapeDtypeStruct((NC, NPAD, 128), jnp.float32),
    )(h, h, W2q, dinv)


def _head_body(h0_ref, h1_ref, wi_ref, bi_ref, wo_ref, bo_ref, o_ref):
    t = jnp.maximum(
        jnp.dot(h0_ref[0], wi_ref[0], preferred_element_type=jnp.float32)
        + jnp.dot(h1_ref[0], wi_ref[1], preferred_element_type=jnp.float32)
        + bi_ref[...], 0.0)
    o_ref[...] = jnp.dot(t, wo_ref[...], preferred_element_type=jnp.float32) \
        + bo_ref[...]


def _head(h, Wiq, bi, Wo, bo):
    blk = 1024
    return pl.pallas_call(
        _head_body,
        grid=(NPAD // blk,),
        in_specs=[
            pl.BlockSpec((1, blk, 128), lambda i: (0, i, 0)),
            pl.BlockSpec((1, blk, 128), lambda i: (1, i, 0)),
            pl.BlockSpec((2, 128, 128), lambda i: (0, 0, 0)),
            pl.BlockSpec((128,), lambda i: (0,)),
            pl.BlockSpec((128, 1), lambda i: (0, 0)),
            pl.BlockSpec((1,), lambda i: (0,)),
        ],
        out_specs=pl.BlockSpec((blk, 1), lambda i: (i, 0)),
        out_shape=jax.ShapeDtypeStruct((NPAD, 1), jnp.float32),
    )(h, h, Wiq, bi, Wo, bo)


def kernel(x, edge_index, edge_attr, W1, b1, W2, b2, Wi, bi, Wo, bo):
    row = edge_index[0]
    col = edge_index[1]
    xp = jnp.pad(x, ((0, NPAD - N), (0, 0)))
    zeros = jnp.zeros((NPAD, 128), jnp.float32)

    W1h = jnp.stack([W1[:, :128], W1[:, 128:]])
    W2q = W2.reshape(2, 128, 2, 128).transpose(0, 2, 1, 3)
    Wiq = Wi.reshape(2, 128, 128)
    b1h = b1.reshape(2, 128)
    b2h = b2.reshape(2, 128)

    degp = _deg(col, edge_attr, zeros)
    y1, dinv = _mm1(xp, W1h, degp)
    h1 = _layer(y1, row, col, edge_attr, dinv, b1h, zeros)
    y2 = _mm2(h1, W2q, dinv)
    h2 = _layer(y2, row, col, edge_attr, dinv, b2h, zeros)
    out = _head(h2, Wiq, bi, Wo, bo)
    return out[:N]


# speedup vs baseline: 8.3123x; 1.3505x over previous
import functools

import jax
import jax.numpy as jnp
from jax import lax
from jax.experimental import pallas as pl
from jax.experimental.pallas import tpu as pltpu
from jax.experimental.pallas import tpu_sc as plsc

N = 10000
NPAD = 10240
E = 320000
NC = 2
NS = 16
CH = 80
WB = 80
EPT = E // NS
NPT = NPAD // NS

_mesh = plsc.VectorSubcoreMesh(core_axis_name="c", subcore_axis_name="s")


def _deg_body(col_hbm, w_hbm, zeros_hbm, deg_hbm, colv, wv, w16, obuf, deg_sh):
    c = lax.axis_index("c")
    s = lax.axis_index("s")
    r0 = s * NPT
    pltpu.sync_copy(zeros_hbm.at[pl.ds(r0, NPT)], deg_sh.at[pl.ds(r0, NPT)])
    plsc.subcore_barrier()

    base0 = c * (E // NC) + s * (E // NC // NS)
    nchunks = E // NC // NS // CH

    def chunk(i, _):
        base = base0 + i * CH
        pltpu.sync_copy(col_hbm.at[pl.ds(base, CH)], colv.at[0])
        pltpu.sync_copy(w_hbm.at[pl.ds(base, CH)], wv)
        for g in range(CH // 16):
            wvec = wv[pl.ds(g * 16, 16)]
            for kk in range(16):
                w16[g * 16 + kk, pl.ds(0, 16)] = jnp.broadcast_to(wvec[kk], (16,))
        pltpu.sync_copy(w16, deg_sh.at[colv.at[0]], add=True)
        return _

    lax.fori_loop(0, nchunks, chunk, None)
    plsc.subcore_barrier()

    for t in range(NPT // 128):
        pltpu.sync_copy(deg_sh.at[pl.ds(r0 + t * 128, 128)], obuf)
        pltpu.sync_copy(obuf, deg_hbm.at[c].at[pl.ds(r0 + t * 128, 128)])


def _deg(col, w, zeros):
    f = functools.partial(
        pl.kernel,
        out_type=jax.ShapeDtypeStruct((NC, NPAD, 128), jnp.float32),
        mesh=_mesh,
        scratch_types=[
            pltpu.VMEM((1, CH), jnp.int32),
            pltpu.VMEM((CH,), jnp.float32),
            pltpu.VMEM((CH, 128), jnp.float32),
            pltpu.VMEM((128, 128), jnp.float32),
            pltpu.VMEM_SHARED((NPAD, 128), jnp.float32),
        ],
    )(_deg_body)
    return f(col, w, zeros)


def _layer_body(y_hbm, row_hbm, col_hbm, w_hbm, dinv_hbm, b_hbm, zeros_hbm,
                h_hbm, rowv0, rowv1, colv0, colv1, wv0, wv1, buf0, buf1,
                hbuf, dvbuf, bbuf, semg0, semg1, sems0, sems1,
                acc_sh):
    abuf = buf0
    ybuf = buf1
    rowv = (rowv0, rowv1)
    colv = (colv0, colv1)
    wv = (wv0, wv1)
    buf = (buf0, buf1)
    semg = (semg0, semg1)
    sems = (sems0, sems1)

    c = lax.axis_index("c")
    s = lax.axis_index("s")
    r0 = s * NPT
    pltpu.sync_copy(zeros_hbm.at[pl.ds(r0, NPT)], acc_sh.at[pl.ds(r0, NPT)])
    pltpu.sync_copy(b_hbm.at[c], bbuf)
    plsc.subcore_barrier()

    base0 = s * EPT
    n = EPT // CH
    yc = y_hbm.at[c]

    def load_idx(j, o):
        base = base0 + j * CH
        pltpu.sync_copy(row_hbm.at[pl.ds(base, CH)], rowv[o])
        pltpu.sync_copy(col_hbm.at[pl.ds(base, CH)], colv[o].at[0])
        pltpu.sync_copy(w_hbm.at[pl.ds(base, CH)], wv[o])

    def scale(b):
        def edge_group(g, carry):
            wvec = wv[b][pl.ds(g * 16, 16)]
            for kk in range(16):
                bw = jnp.broadcast_to(wvec[kk], (16,))
                for j in range(8):
                    sl = pl.ds(j * 16, 16)
                    buf[b][g * 16 + kk, sl] = buf[b][g * 16 + kk, sl] * bw
            return carry

        lax.fori_loop(0, CH // 16, edge_group, None)

    load_idx(0, 0)
    g0 = pltpu.async_copy(yc.at[rowv0], buf0, semg0)

    def pair(i2, _):
        for b in range(2):
            i = 2 * i2 + b
            o = 1 - b

            @pl.when(i <= n - 2)
            def _():
                load_idx(i + 1, o)
                pltpu.async_copy(yc.at[rowv[o]], buf[o], semg[o])

            pltpu.make_async_copy(yc.at[rowv[b]], buf[b], semg[b]).wait()
            scale(b)
            pltpu.sync_copy(buf[b], acc_sh.at[colv[b].at[0]], add=True)
        return _

    lax.fori_loop(0, n // 2, pair, None)
    plsc.subcore_barrier()

    for t in range(NPT // WB):
        w0 = r0 + t * WB
        pltpu.sync_copy(acc_sh.at[pl.ds(w0, WB)], abuf)
        pltpu.sync_copy(yc.at[pl.ds(w0, WB)], ybuf)
        pltpu.sync_copy(dinv_hbm.at[pl.ds(w0, WB)], dvbuf)

        def node_group(g, _):
            dvec = dvbuf[pl.ds(g * 16, 16)]
            for kk in range(16):
                dv = jnp.broadcast_to(dvec[kk], (16,))
                for j in range(8):
                    sl = pl.ds(j * 16, 16)
                    hbuf[g * 16 + kk, sl] = jnp.maximum(
                        (abuf[g * 16 + kk, sl] + ybuf[g * 16 + kk, sl]) * dv
                        + bbuf[sl], 0.0)
            return _

        lax.fori_loop(0, WB // 16, node_group, None)
        pltpu.sync_copy(hbuf, h_hbm.at[c].at[pl.ds(w0, WB)])


def _layer(y, row, col, w, dinv, bh, zeros):
    f = functools.partial(
        pl.kernel,
        out_type=jax.ShapeDtypeStruct((NC, NPAD, 128), jnp.float32),
        mesh=_mesh,
        scratch_types=[
            pltpu.VMEM((CH,), jnp.int32),
            pltpu.VMEM((CH,), jnp.int32),
            pltpu.VMEM((1, CH), jnp.int32),
            pltpu.VMEM((1, CH), jnp.int32),
            pltpu.VMEM((CH,), jnp.float32),
            pltpu.VMEM((CH,), jnp.float32),
            pltpu.VMEM((CH, 128), jnp.float32),
            pltpu.VMEM((CH, 128), jnp.float32),
            pltpu.VMEM((WB, 128), jnp.float32),
            pltpu.VMEM((WB,), jnp.float32),
            pltpu.VMEM((128,), jnp.float32),
            pltpu.SemaphoreType.DMA,
            pltpu.SemaphoreType.DMA,
            pltpu.SemaphoreType.DMA,
            pltpu.SemaphoreType.DMA,
            pltpu.VMEM_SHARED((NPAD, 128), jnp.float32),
        ],
    )(_layer_body)
    return f(y, row, col, w, dinv, bh, zeros)


def _mm1_body(x_ref, wh_ref, d0_ref, d1_ref, y_ref, dinv_ref):
    deg = d0_ref[0, :, :1] + d1_ref[0, :, :1] + 1.0
    dinv = lax.rsqrt(deg)
    x = x_ref[...]
    y_ref[0] = jnp.dot(x, wh_ref[0], preferred_element_type=jnp.float32) * dinv
    y_ref[1] = jnp.dot(x, wh_ref[1], preferred_element_type=jnp.float32) * dinv
    dinv_ref[...] = dinv[:, 0]


def _mm1(xp, Wh, degp):
    blk = 1024
    return pl.pallas_call(
        _mm1_body,
        grid=(NPAD // blk,),
        in_specs=[
            pl.BlockSpec((blk, 128), lambda i: (i, 0)),
            pl.BlockSpec((2, 128, 128), lambda i: (0, 0, 0)),
            pl.BlockSpec((1, blk, 128), lambda i: (0, i, 0)),
            pl.BlockSpec((1, blk, 128), lambda i: (1, i, 0)),
        ],
        out_specs=[
            pl.BlockSpec((2, blk, 128), lambda i: (0, i, 0)),
            pl.BlockSpec((blk,), lambda i: (i,)),
        ],
        out_shape=[
            jax.ShapeDtypeStruct((NC, NPAD, 128), jnp.float32),
            jax.ShapeDtypeStruct((NPAD,), jnp.float32),
        ],
    )(xp, Wh, degp, degp)


def _mm2_body(h0_ref, h1_ref, w_ref, dinv_ref, y_ref):
    dinv = dinv_ref[...][:, None]
    h0 = h0_ref[0]
    h1 = h1_ref[0]
    for c in range(2):
        xw = (jnp.dot(h0, w_ref[0, c], preferred_element_type=jnp.float32)
              + jnp.dot(h1, w_ref[1, c], preferred_element_type=jnp.float32))
        y_ref[c] = xw * dinv


def _mm2(h, W2q, dinv):
    blk = 1024
    return pl.pallas_call(
        _mm2_body,
        grid=(NPAD // blk,),
        in_specs=[
            pl.BlockSpec((1, blk, 128), lambda i: (0, i, 0)),
            pl.BlockSpec((1, blk, 128), lambda i: (1, i, 0)),
            pl.BlockSpec((2, 2, 128, 128), lambda i: (0, 0, 0, 0)),
            pl.BlockSpec((blk,), lambda i: (i,)),
        ],
        out_specs=pl.BlockSpec((2, blk, 128), lambda i: (0, i, 0)),
        out_shape=jax.ShapeDtypeStruct((NC, NPAD, 128), jnp.float32),
    )(h, h, W2q, dinv)


def _head_body(h0_ref, h1_ref, wi_ref, bi_ref, wo_ref, bo_ref, o_ref):
    t = jnp.maximum(
        jnp.dot(h0_ref[0], wi_ref[0], preferred_element_type=jnp.float32)
        + jnp.dot(h1_ref[0], wi_ref[1], preferred_element_type=jnp.float32)
        + bi_ref[...], 0.0)
    o_ref[...] = jnp.dot(t, wo_ref[...], preferred_element_type=jnp.float32) \
        + bo_ref[...]


def _head(h, Wiq, bi, Wo, bo):
    blk = 1024
    return pl.pallas_call(
        _head_body,
        grid=(NPAD // blk,),
        in_specs=[
            pl.BlockSpec((1, blk, 128), lambda i: (0, i, 0)),
            pl.BlockSpec((1, blk, 128), lambda i: (1, i, 0)),
            pl.BlockSpec((2, 128, 128), lambda i: (0, 0, 0)),
            pl.BlockSpec((128,), lambda i: (0,)),
            pl.BlockSpec((128, 1), lambda i: (0, 0)),
            pl.BlockSpec((1,), lambda i: (0,)),
        ],
        out_specs=pl.BlockSpec((blk, 1), lambda i: (i, 0)),
        out_shape=jax.ShapeDtypeStruct((NPAD, 1), jnp.float32),
    )(h, h, Wiq, bi, Wo, bo)


def kernel(x, edge_index, edge_attr, W1, b1, W2, b2, Wi, bi, Wo, bo):
    row = edge_index[0]
    col = edge_index[1]
    xp = jnp.pad(x, ((0, NPAD - N), (0, 0)))
    zeros = jnp.zeros((NPAD, 128), jnp.float32)

    W1h = jnp.stack([W1[:, :128], W1[:, 128:]])
    W2q = W2.reshape(2, 128, 2, 128).transpose(0, 2, 1, 3)
    Wiq = Wi.reshape(2, 128, 128)
    b1h = b1.reshape(2, 128)
    b2h = b2.reshape(2, 128)

    degp = _deg(col, edge_attr, zeros)
    y1, dinv = _mm1(xp, W1h, degp)
    h1 = _layer(y1, row, col, edge_attr, dinv, b1h, zeros)
    y2 = _mm2(h1, W2q, dinv)
    h2 = _layer(y2, row, col, edge_attr, dinv, b2h, zeros)
    out = _head(h2, Wiq, bi, Wo, bo)
    return out[:N]


# speedup vs baseline: 8.3425x; 1.0036x over previous
import functools

import jax
import jax.numpy as jnp
from jax import lax
from jax.experimental import pallas as pl
from jax.experimental.pallas import tpu as pltpu
from jax.experimental.pallas import tpu_sc as plsc

N = 10000
NPAD = 10240
E = 320000
NC = 2
NS = 16
CH = 80
WB = 80
EPT = E // NS
NPT = NPAD // NS

_mesh = plsc.VectorSubcoreMesh(core_axis_name="c", subcore_axis_name="s")


def _deg_body(col_hbm, w_hbm, zeros_hbm, deg_hbm, colv0, colv1, wv0, wv1,
              w16a, w16b, obuf, sems0, sems1, deg_sh):
    colv = (colv0, colv1)
    wv = (wv0, wv1)
    w16 = (w16a, w16b)
    sems = (sems0, sems1)
    c = lax.axis_index("c")
    s = lax.axis_index("s")
    r0 = s * NPT
    pltpu.sync_copy(zeros_hbm.at[pl.ds(r0, NPT)], deg_sh.at[pl.ds(r0, NPT)])
    plsc.subcore_barrier()

    base0 = c * (E // NC) + s * (E // NC // NS)
    n = E // NC // NS // CH

    def load_idx(j, o):
        base = base0 + j * CH
        pltpu.sync_copy(col_hbm.at[pl.ds(base, CH)], colv[o].at[0])
        pltpu.sync_copy(w_hbm.at[pl.ds(base, CH)], wv[o])

    load_idx(0, 0)

    def pair(i2, _):
        for b in range(2):
            i = 2 * i2 + b
            o = 1 - b

            @pl.when(i >= 1)
            def _():
                pltpu.make_async_copy(
                    w16[o], deg_sh.at[colv[o].at[0]], sems[o]).wait()

            @pl.when(i <= n - 2)
            def _():
                load_idx(i + 1, o)

            def build(g, carry):
                wvec = wv[b][pl.ds(g * 16, 16)]
                for kk in range(16):
                    w16[b][g * 16 + kk, pl.ds(0, 16)] = jnp.broadcast_to(
                        wvec[kk], (16,))
                return carry

            lax.fori_loop(0, CH // 16, build, None)
            pltpu.async_copy(w16[b], deg_sh.at[colv[b].at[0]], sems[b],
                             add=True)
        return _

    lax.fori_loop(0, n // 2, pair, None)
    pltpu.make_async_copy(w16b, deg_sh.at[colv1.at[0]], sems1).wait()
    plsc.subcore_barrier()

    for t in range(NPT // 128):
        pltpu.sync_copy(deg_sh.at[pl.ds(r0 + t * 128, 128)], obuf)
        pltpu.sync_copy(obuf, deg_hbm.at[c].at[pl.ds(r0 + t * 128, 128)])


def _deg(col, w, zeros):
    f = functools.partial(
        pl.kernel,
        out_type=jax.ShapeDtypeStruct((NC, NPAD, 128), jnp.float32),
        mesh=_mesh,
        scratch_types=[
            pltpu.VMEM((1, CH), jnp.int32),
            pltpu.VMEM((1, CH), jnp.int32),
            pltpu.VMEM((CH,), jnp.float32),
            pltpu.VMEM((CH,), jnp.float32),
            pltpu.VMEM((CH, 128), jnp.float32),
            pltpu.VMEM((CH, 128), jnp.float32),
            pltpu.VMEM((128, 128), jnp.float32),
            pltpu.SemaphoreType.DMA,
            pltpu.SemaphoreType.DMA,
            pltpu.VMEM_SHARED((NPAD, 128), jnp.float32),
        ],
    )(_deg_body)
    return f(col, w, zeros)


def _layer_body(y_hbm, row_hbm, col_hbm, w_hbm, dinv_hbm, b_hbm, zeros_hbm,
                h_hbm, rowv0, rowv1, colv0, colv1, wv0, wv1, buf0, buf1,
                hbuf, dvbuf, bbuf, semg0, semg1, sems0, sems1,
                acc_sh):
    abuf = buf0
    ybuf = buf1
    rowv = (rowv0, rowv1)
    colv = (colv0, colv1)
    wv = (wv0, wv1)
    buf = (buf0, buf1)
    semg = (semg0, semg1)
    sems = (sems0, sems1)

    c = lax.axis_index("c")
    s = lax.axis_index("s")
    r0 = s * NPT
    pltpu.sync_copy(zeros_hbm.at[pl.ds(r0, NPT)], acc_sh.at[pl.ds(r0, NPT)])
    pltpu.sync_copy(b_hbm.at[c], bbuf)
    plsc.subcore_barrier()

    base0 = s * EPT
    n = EPT // CH
    yc = y_hbm.at[c]

    def load_idx(j, o):
        base = base0 + j * CH
        pltpu.sync_copy(row_hbm.at[pl.ds(base, CH)], rowv[o])
        pltpu.sync_copy(col_hbm.at[pl.ds(base, CH)], colv[o].at[0])
        pltpu.sync_copy(w_hbm.at[pl.ds(base, CH)], wv[o])

    def scale(b):
        def edge_group(g, carry):
            wvec = wv[b][pl.ds(g * 16, 16)]
            for kk in range(16):
                bw = jnp.broadcast_to(wvec[kk], (16,))
                for j in range(8):
                    sl = pl.ds(j * 16, 16)
                    buf[b][g * 16 + kk, sl] = buf[b][g * 16 + kk, sl] * bw
            return carry

        lax.fori_loop(0, CH // 16, edge_group, None)

    load_idx(0, 0)
    g0 = pltpu.async_copy(yc.at[rowv0], buf0, semg0)

    def pair(i2, _):
        for b in range(2):
            i = 2 * i2 + b
            o = 1 - b

            @pl.when(i >= 1)
            def _():
                pltpu.make_async_copy(
                    buf[o], acc_sh.at[colv[o].at[0]], sems[o]).wait()

            @pl.when(i <= n - 2)
            def _():
                load_idx(i + 1, o)
                pltpu.async_copy(yc.at[rowv[o]], buf[o], semg[o])

            pltpu.make_async_copy(yc.at[rowv[b]], buf[b], semg[b]).wait()
            scale(b)
            pltpu.async_copy(buf[b], acc_sh.at[colv[b].at[0]], sems[b],
                             add=True)
        return _

    lax.fori_loop(0, n // 2, pair, None)
    pltpu.make_async_copy(buf1, acc_sh.at[colv1.at[0]], sems1).wait()
    plsc.subcore_barrier()

    for t in range(NPT // WB):
        w0 = r0 + t * WB
        pltpu.sync_copy(acc_sh.at[pl.ds(w0, WB)], abuf)
        pltpu.sync_copy(yc.at[pl.ds(w0, WB)], ybuf)
        pltpu.sync_copy(dinv_hbm.at[pl.ds(w0, WB)], dvbuf)

        def node_group(g, _):
            dvec = dvbuf[pl.ds(g * 16, 16)]
            for kk in range(16):
                dv = jnp.broadcast_to(dvec[kk], (16,))
                for j in range(8):
                    sl = pl.ds(j * 16, 16)
                    hbuf[g * 16 + kk, sl] = jnp.maximum(
                        (abuf[g * 16 + kk, sl] + ybuf[g * 16 + kk, sl]) * dv
                        + bbuf[sl], 0.0)
            return _

        lax.fori_loop(0, WB // 16, node_group, None)
        pltpu.sync_copy(hbuf, h_hbm.at[c].at[pl.ds(w0, WB)])


def _layer(y, row, col, w, dinv, bh, zeros):
    f = functools.partial(
        pl.kernel,
        out_type=jax.ShapeDtypeStruct((NC, NPAD, 128), jnp.float32),
        mesh=_mesh,
        scratch_types=[
            pltpu.VMEM((CH,), jnp.int32),
            pltpu.VMEM((CH,), jnp.int32),
            pltpu.VMEM((1, CH), jnp.int32),
            pltpu.VMEM((1, CH), jnp.int32),
            pltpu.VMEM((CH,), jnp.float32),
            pltpu.VMEM((CH,), jnp.float32),
            pltpu.VMEM((CH, 128), jnp.float32),
            pltpu.VMEM((CH, 128), jnp.float32),
            pltpu.VMEM((WB, 128), jnp.float32),
            pltpu.VMEM((WB,), jnp.float32),
            pltpu.VMEM((128,), jnp.float32),
            pltpu.SemaphoreType.DMA,
            pltpu.SemaphoreType.DMA,
            pltpu.SemaphoreType.DMA,
            pltpu.SemaphoreType.DMA,
            pltpu.VMEM_SHARED((NPAD, 128), jnp.float32),
        ],
    )(_layer_body)
    return f(y, row, col, w, dinv, bh, zeros)


def _mm1_body(x_ref, wh_ref, d0_ref, d1_ref, y_ref, dinv_ref):
    deg = d0_ref[0, :, :1] + d1_ref[0, :, :1] + 1.0
    dinv = lax.rsqrt(deg)
    x = x_ref[...]
    y_ref[0] = jnp.dot(x, wh_ref[0], preferred_element_type=jnp.float32) * dinv
    y_ref[1] = jnp.dot(x, wh_ref[1], preferred_element_type=jnp.float32) * dinv
    dinv_ref[...] = dinv[:, 0]


def _mm1(xp, Wh, degp):
    blk = 1024
    return pl.pallas_call(
        _mm1_body,
        grid=(NPAD // blk,),
        in_specs=[
            pl.BlockSpec((blk, 128), lambda i: (i, 0)),
            pl.BlockSpec((2, 128, 128), lambda i: (0, 0, 0)),
            pl.BlockSpec((1, blk, 128), lambda i: (0, i, 0)),
            pl.BlockSpec((1, blk, 128), lambda i: (1, i, 0)),
        ],
        out_specs=[
            pl.BlockSpec((2, blk, 128), lambda i: (0, i, 0)),
            pl.BlockSpec((blk,), lambda i: (i,)),
        ],
        out_shape=[
            jax.ShapeDtypeStruct((NC, NPAD, 128), jnp.float32),
            jax.ShapeDtypeStruct((NPAD,), jnp.float32),
        ],
    )(xp, Wh, degp, degp)


def _mm2_body(h0_ref, h1_ref, w_ref, dinv_ref, y_ref):
    dinv = dinv_ref[...][:, None]
    h0 = h0_ref[0]
    h1 = h1_ref[0]
    for c in range(2):
        xw = (jnp.dot(h0, w_ref[0, c], preferred_element_type=jnp.float32)
              + jnp.dot(h1, w_ref[1, c], preferred_element_type=jnp.float32))
        y_ref[c] = xw * dinv


def _mm2(h, W2q, dinv):
    blk = 1024
    return pl.pallas_call(
        _mm2_body,
        grid=(NPAD // blk,),
        in_specs=[
            pl.BlockSpec((1, blk, 128), lambda i: (0, i, 0)),
            pl.BlockSpec((1, blk, 128), lambda i: (1, i, 0)),
            pl.BlockSpec((2, 2, 128, 128), lambda i: (0, 0, 0, 0)),
            pl.BlockSpec((blk,), lambda i: (i,)),
        ],
        out_specs=pl.BlockSpec((2, blk, 128), lambda i: (0, i, 0)),
        out_shape=jax.ShapeDtypeStruct((NC, NPAD, 128), jnp.float32),
    )(h, h, W2q, dinv)


def _head_body(h0_ref, h1_ref, wi_ref, bi_ref, wo_ref, bo_ref, o_ref):
    t = jnp.maximum(
        jnp.dot(h0_ref[0], wi_ref[0], preferred_element_type=jnp.float32)
        + jnp.dot(h1_ref[0], wi_ref[1], preferred_element_type=jnp.float32)
        + bi_ref[...], 0.0)
    o_ref[...] = jnp.dot(t, wo_ref[...], preferred_element_type=jnp.float32) \
        + bo_ref[...]


def _head(h, Wiq, bi, Wo, bo):
    blk = 1024
    return pl.pallas_call(
        _head_body,
        grid=(NPAD // blk,),
        in_specs=[
            pl.BlockSpec((1, blk, 128), lambda i: (0, i, 0)),
            pl.BlockSpec((1, blk, 128), lambda i: (1, i, 0)),
            pl.BlockSpec((2, 128, 128), lambda i: (0, 0, 0)),
            pl.BlockSpec((128,), lambda i: (0,)),
            pl.BlockSpec((128, 1), lambda i: (0, 0)),
            pl.BlockSpec((1,), lambda i: (0,)),
        ],
        out_specs=pl.BlockSpec((blk, 1), lambda i: (i, 0)),
        out_shape=jax.ShapeDtypeStruct((NPAD, 1), jnp.float32),
    )(h, h, Wiq, bi, Wo, bo)


def kernel(x, edge_index, edge_attr, W1, b1, W2, b2, Wi, bi, Wo, bo):
    row = edge_index[0]
    col = edge_index[1]
    xp = jnp.pad(x, ((0, NPAD - N), (0, 0)))
    zeros = jnp.zeros((NPAD, 128), jnp.float32)

    W1h = jnp.stack([W1[:, :128], W1[:, 128:]])
    W2q = W2.reshape(2, 128, 2, 128).transpose(0, 2, 1, 3)
    Wiq = Wi.reshape(2, 128, 128)
    b1h = b1.reshape(2, 128)
    b2h = b2.reshape(2, 128)

    degp = _deg(col, edge_attr, zeros)
    y1, dinv = _mm1(xp, W1h, degp)
    h1 = _layer(y1, row, col, edge_attr, dinv, b1h, zeros)
    y2 = _mm2(h1, W2q, dinv)
    h2 = _layer(y2, row, col, edge_attr, dinv, b2h, zeros)
    out = _head(h2, Wiq, bi, Wo, bo)
    return out[:N]
